# Initial kernel scaffold; baseline (speedup 1.0000x reference)
import jax
import jax.numpy as jnp
from jax.experimental import pallas as pl


def kernel(x, edge_index, batch, params):
    raise NotImplementedError("write your pallas kernel here")


# speedup vs baseline: 5.8669x; 5.8669x over previous
import functools

import jax
import jax.numpy as jnp
from jax import lax
from jax.experimental import pallas as pl
from jax.experimental.pallas import tpu as pltpu
from jax.experimental.pallas import tpu_sc as plsc

_NC = 2
_NS = 16


@functools.partial(jax.jit, static_argnames=("n", "e", "d"))
def _sc_scatter_add(h, src, dst, *, n, e, d):
    nw = _NC * _NS
    epw = e // nw
    ch = 128
    n_full = epw // ch
    tail = epw - n_full * ch
    rpt = (n // _NS) // 8 * 8
    zr = 208
    assert rpt % zr == 0 and tail % 8 == 0 and (n - 16 * rpt) % 8 == 0

    mesh = plsc.VectorSubcoreMesh(core_axis_name="c", subcore_axis_name="s")

    scratch = [
        pltpu.VMEM((ch,), jnp.int32),
        pltpu.VMEM((ch,), jnp.int32),
        pltpu.VMEM((ch, d), jnp.float32),
        pltpu.VMEM((zr, d), jnp.float32),
        pltpu.VMEM_SHARED((n, d), jnp.float32),
        pltpu.SemaphoreType.DMA,
    ]
    if tail:
        scratch += [
            pltpu.VMEM((tail,), jnp.int32),
            pltpu.VMEM((tail,), jnp.int32),
            pltpu.VMEM((tail, d), jnp.float32),
        ]

    @functools.partial(
        pl.kernel,
        out_type=jax.ShapeDtypeStruct((_NC * n, d), jnp.float32),
        mesh=mesh,
        scratch_types=scratch,
    )
    def k(h_hbm, src_hbm, dst_hbm, out_hbm, src_v, dst_v, rows_v, zbuf,
          agg_sh, sem, *tail_refs):
        cid = lax.axis_index("c")
        sid = lax.axis_index("s")
        wid = cid * _NS + sid
        base0 = wid * epw

        zeros16 = jnp.zeros((16,), jnp.float32)

        def zrow(r, _):
            for j in range(d // 16):
                zbuf[r, pl.ds(j * 16, 16)] = zeros16
            return 0

        lax.fori_loop(0, zr, zrow, 0)
        for i in range(rpt // zr):
            pltpu.sync_copy(zbuf, agg_sh.at[pl.ds(sid * rpt + i * zr, zr)])
        nrem = n - _NS * rpt

        @pl.when(sid == _NS - 1)
        def _():
            pltpu.sync_copy(zbuf.at[pl.ds(0, nrem)],
                            agg_sh.at[pl.ds(_NS * rpt, nrem)])

        plsc.subcore_barrier()

        def chunk(c, _):
            base = base0 + c * ch
            pltpu.sync_copy(src_hbm.at[pl.ds(base, ch)], src_v)
            pltpu.sync_copy(dst_hbm.at[pl.ds(base, ch)], dst_v)
            pltpu.async_copy(h_hbm.at[src_v], rows_v, sem).wait()
            pltpu.sync_copy(rows_v, agg_sh.at[dst_v], add=True)
            return 0

        lax.fori_loop(0, n_full, chunk, 0)

        if tail:
            src_t, dst_t, rows_t = tail_refs
            base = base0 + n_full * ch
            pltpu.sync_copy(src_hbm.at[pl.ds(base, tail)], src_t)
            pltpu.sync_copy(dst_hbm.at[pl.ds(base, tail)], dst_t)
            pltpu.async_copy(h_hbm.at[src_t], rows_t, sem).wait()
            pltpu.sync_copy(rows_t, agg_sh.at[dst_t], add=True)

        plsc.subcore_barrier()

        pltpu.sync_copy(
            agg_sh.at[pl.ds(sid * rpt, rpt)],
            out_hbm.at[pl.ds(cid * n + sid * rpt, rpt)],
        )

        @pl.when(sid == _NS - 1)
        def _():
            pltpu.sync_copy(
                agg_sh.at[pl.ds(_NS * rpt, nrem)],
                out_hbm.at[pl.ds(cid * n + _NS * rpt, nrem)],
            )

    return k(h, src, dst)


def _tc_layer(h, agg, batch, p, *, n, d, hdim, g):
    eps = 1e-5

    def body(h_ref, agg_ref, b_ref, w1, b1, g1, be1, w2, b2, g2, be2,
             hout_ref, pool_ref):
        y = h_ref[...] + agg_ref[pl.ds(0, n), :] + agg_ref[pl.ds(n, n), :]
        z = jnp.dot(y, w1[...], preferred_element_type=jnp.float32) + b1[...]
        m = jnp.mean(z, axis=0)
        v = jnp.mean(z * z, axis=0) - m * m
        z = g1[...] * (z - m) * lax.rsqrt(v + eps) + be1[...]
        z = jnp.maximum(z, 0.0)
        z = jnp.dot(z, w2[...], preferred_element_type=jnp.float32) + b2[...]
        m2 = jnp.mean(z, axis=0)
        v2 = jnp.mean(z * z, axis=0) - m2 * m2
        z = g2[...] * (z - m2) * lax.rsqrt(v2 + eps) + be2[...]
        hn = jnp.maximum(z, 0.0)
        hout_ref[...] = hn
        seg = lax.broadcasted_iota(jnp.int32, (g, n), 0)
        onehot = (seg == b_ref[...][None, :]).astype(jnp.float32)
        pool_ref[...] = jnp.dot(onehot, hn, preferred_element_type=jnp.float32)

    return pl.pallas_call(
        body,
        out_shape=(
            jax.ShapeDtypeStruct((n, hdim), jnp.float32),
            jax.ShapeDtypeStruct((g, hdim), jnp.float32),
        ),
    )(h, agg, batch, p["W1"], p["b1"], p["g1"], p["be1"],
      p["W2"], p["b2"], p["g2"], p["be2"])


def kernel(x, edge_index, batch, params):
    n, d = x.shape
    e = edge_index.shape[1]
    g = 64
    src = edge_index[0]
    dst = edge_index[1]
    h = x
    pooled = []
    for p in params:
        hdim = p["W2"].shape[1]
        agg = _sc_scatter_add(h, src, dst, n=n, e=e, d=h.shape[1])
        h, pool = _tc_layer(h, agg, batch, p, n=n, d=h.shape[1], hdim=hdim, g=g)
        pooled.append(pool)
    return jnp.concatenate(pooled, axis=-1)


# speedup vs baseline: 8.4485x; 1.4400x over previous
import functools

import jax
import jax.numpy as jnp
from jax import lax
from jax.experimental import pallas as pl
from jax.experimental.pallas import tpu as pltpu
from jax.experimental.pallas import tpu_sc as plsc

_NC = 2
_NS = 16


@functools.partial(jax.jit, static_argnames=("n", "e", "d"))
def _sc_scatter_add(h, src, dst, *, n, e, d):
    nw = _NC * _NS
    epw = e // nw
    ch = 128
    n_full = epw // ch
    tail = epw - n_full * ch
    assert n_full % 2 == 0 and tail % 8 == 0
    rpt = (n // _NS) // 8 * 8
    nrem = n - _NS * rpt

    mesh = plsc.VectorSubcoreMesh(core_axis_name="c", subcore_axis_name="s")

    scratch = [
        pltpu.VMEM((ch,), jnp.int32),
        pltpu.VMEM((ch,), jnp.int32),
        pltpu.VMEM((ch,), jnp.int32),
        pltpu.VMEM((ch,), jnp.int32),
        pltpu.VMEM((ch, d), jnp.float32),
        pltpu.VMEM((ch, d), jnp.float32),
        pltpu.VMEM((tail,), jnp.int32),
        pltpu.VMEM((tail,), jnp.int32),
        pltpu.VMEM((tail, d), jnp.float32),
        pltpu.VMEM_SHARED((n, d), jnp.float32),
        pltpu.SemaphoreType.DMA,
        pltpu.SemaphoreType.DMA,
        pltpu.SemaphoreType.DMA,
        pltpu.SemaphoreType.DMA,
    ]

    @functools.partial(
        pl.kernel,
        out_type=jax.ShapeDtypeStruct((_NC * n, d), jnp.float32),
        mesh=mesh,
        scratch_types=scratch,
    )
    def k(h_hbm, src_hbm, dst_hbm, out_hbm, src0, src1, dst0, dst1,
          rows0, rows1, src_t, dst_t, rows_t, agg_sh, gsem, isem,
          ssem0, ssem1):
        cid = lax.axis_index("c")
        sid = lax.axis_index("s")
        wid = cid * _NS + sid
        base0 = wid * epw
        srcb, dstb, rowsb, ssem = (src0, src1), (dst0, dst1), (rows0, rows1), \
            (ssem0, ssem1)

        zeros16 = jnp.zeros((16,), jnp.float32)

        def zrow(r, _):
            for j in range(d // 16):
                rows0[r, pl.ds(j * 16, 16)] = zeros16
            return 0

        lax.fori_loop(0, ch, zrow, 0)
        nfull_z = rpt // ch
        for i in range(nfull_z):
            pltpu.sync_copy(rows0, agg_sh.at[pl.ds(sid * rpt + i * ch, ch)])
        zrem = rpt - nfull_z * ch
        if zrem:
            pltpu.sync_copy(rows0.at[pl.ds(0, zrem)],
                            agg_sh.at[pl.ds(sid * rpt + nfull_z * ch, zrem)])

        @pl.when(sid == _NS - 1)
        def _():
            pltpu.sync_copy(rows0.at[pl.ds(0, nrem)],
                            agg_sh.at[pl.ds(_NS * rpt, nrem)])

        plsc.subcore_barrier()

        def drain(b):
            pltpu.make_async_copy(rowsb[b], agg_sh.at[dstb[b]],
                                  ssem[b]).wait()

        def chunk(c, b, first):
            base = base0 + c * ch
            if not first:
                drain(b)
            di = pltpu.async_copy(src_hbm.at[pl.ds(base, ch)], srcb[b], isem)
            dj = pltpu.async_copy(dst_hbm.at[pl.ds(base, ch)], dstb[b], isem)
            di.wait()
            dj.wait()
            pltpu.async_copy(h_hbm.at[srcb[b]], rowsb[b], gsem).wait()
            pltpu.async_copy(rowsb[b], agg_sh.at[dstb[b]], ssem[b], add=True)

        def pair(i, _):
            for b in range(2):
                c = 2 * i + b

                @pl.when(i == 0)
                def _():
                    chunk(c, b, True)

                @pl.when(i > 0)
                def _():
                    chunk(c, b, False)
            return 0

        lax.fori_loop(0, n_full // 2, pair, 0)
        drain(0)
        drain(1)

        base = base0 + n_full * ch
        pltpu.sync_copy(src_hbm.at[pl.ds(base, tail)], src_t)
        pltpu.sync_copy(dst_hbm.at[pl.ds(base, tail)], dst_t)
        pltpu.async_copy(h_hbm.at[src_t], rows_t, gsem).wait()
        pltpu.sync_copy(rows_t, agg_sh.at[dst_t], add=True)

        plsc.subcore_barrier()

        pltpu.sync_copy(
            agg_sh.at[pl.ds(sid * rpt, rpt)],
            out_hbm.at[pl.ds(cid * n + sid * rpt, rpt)],
        )

        @pl.when(sid == _NS - 1)
        def _():
            pltpu.sync_copy(
                agg_sh.at[pl.ds(_NS * rpt, nrem)],
                out_hbm.at[pl.ds(cid * n + _NS * rpt, nrem)],
            )

    return k(h, src, dst)


def _tc_layer(h, agg, batch, p, *, n, d, hdim, g):
    eps = 1e-5

    def body(h_ref, agg_ref, b_ref, w1, b1, g1, be1, w2, b2, g2, be2,
             hout_ref, pool_ref):
        y = h_ref[...] + agg_ref[pl.ds(0, n), :] + agg_ref[pl.ds(n, n), :]
        z = jnp.dot(y, w1[...], preferred_element_type=jnp.float32) + b1[...]
        m = jnp.mean(z, axis=0)
        v = jnp.mean(z * z, axis=0) - m * m
        z = g1[...] * (z - m) * lax.rsqrt(v + eps) + be1[...]
        z = jnp.maximum(z, 0.0)
        z = jnp.dot(z, w2[...], preferred_element_type=jnp.float32) + b2[...]
        m2 = jnp.mean(z, axis=0)
        v2 = jnp.mean(z * z, axis=0) - m2 * m2
        z = g2[...] * (z - m2) * lax.rsqrt(v2 + eps) + be2[...]
        hn = jnp.maximum(z, 0.0)
        hout_ref[...] = hn
        seg = lax.broadcasted_iota(jnp.int32, (g, n), 0)
        onehot = (seg == b_ref[...][None, :]).astype(jnp.float32)
        pool_ref[...] = jnp.dot(onehot, hn, preferred_element_type=jnp.float32)

    return pl.pallas_call(
        body,
        out_shape=(
            jax.ShapeDtypeStruct((n, hdim), jnp.float32),
            jax.ShapeDtypeStruct((g, hdim), jnp.float32),
        ),
    )(h, agg, batch, p["W1"], p["b1"], p["g1"], p["be1"],
      p["W2"], p["b2"], p["g2"], p["be2"])


def kernel(x, edge_index, batch, params):
    n, d = x.shape
    e = edge_index.shape[1]
    g = 64
    src = edge_index[0]
    dst = edge_index[1]
    h = x
    pooled = []
    for p in params:
        hdim = p["W2"].shape[1]
        agg = _sc_scatter_add(h, src, dst, n=n, e=e, d=h.shape[1])
        h, pool = _tc_layer(h, agg, batch, p, n=n, d=h.shape[1], hdim=hdim, g=g)
        pooled.append(pool)
    return jnp.concatenate(pooled, axis=-1)


# speedup vs baseline: 10.3944x; 1.2303x over previous
import functools

import jax
import jax.numpy as jnp
from jax import lax
from jax.experimental import pallas as pl
from jax.experimental.pallas import tpu as pltpu
from jax.experimental.pallas import tpu_sc as plsc

_NC = 2
_NS = 16


@functools.partial(jax.jit, static_argnames=("n", "e", "d"))
def _sc_scatter_add(h, src, dst, *, n, e, d):
    nw = _NC * _NS
    epw = e // nw
    ch = 128
    n_full = epw // ch
    tail = epw - n_full * ch
    assert n_full % 2 == 0 and tail % 8 == 0
    rpt = (n // _NS) // 8 * 8
    nrem = n - _NS * rpt

    mesh = plsc.VectorSubcoreMesh(core_axis_name="c", subcore_axis_name="s")

    scratch = (
        [pltpu.VMEM((ch,), jnp.int32)] * 4 +
        [pltpu.VMEM((ch,), jnp.int32)] * 4 +
        [pltpu.VMEM((ch, d), jnp.float32)] * 2 +
        [
            pltpu.VMEM((tail,), jnp.int32),
            pltpu.VMEM((tail,), jnp.int32),
            pltpu.VMEM((tail, d), jnp.float32),
            pltpu.VMEM_SHARED((n, d), jnp.float32),
            pltpu.SemaphoreType.DMA,
        ] +
        [pltpu.SemaphoreType.DMA] * 4 +
        [pltpu.SemaphoreType.DMA] * 2
    )

    @functools.partial(
        pl.kernel,
        out_type=jax.ShapeDtypeStruct((_NC * n, d), jnp.float32),
        mesh=mesh,
        scratch_types=scratch,
    )
    def k(h_hbm, src_hbm, dst_hbm, out_hbm,
          src_a, src_b, src_c, src_d, dst_a, dst_b, dst_c, dst_d,
          rows0, rows1, src_t, dst_t, rows_t, agg_sh, gsem,
          isem_a, isem_b, isem_c, isem_d, ssem0, ssem1):
        cid = lax.axis_index("c")
        sid = lax.axis_index("s")
        wid = cid * _NS + sid
        base0 = wid * epw
        srcb = (src_a, src_b, src_c, src_d)
        dstb = (dst_a, dst_b, dst_c, dst_d)
        isem = (isem_a, isem_b, isem_c, isem_d)
        rowsb = (rows0, rows1)
        ssem = (ssem0, ssem1)

        zeros16 = jnp.zeros((16,), jnp.float32)

        def zrow(r, _):
            for j in range(d // 16):
                rows0[r, pl.ds(j * 16, 16)] = zeros16
            return 0

        lax.fori_loop(0, ch, zrow, 0)
        nfull_z = rpt // ch
        for i in range(nfull_z):
            pltpu.sync_copy(rows0, agg_sh.at[pl.ds(sid * rpt + i * ch, ch)])
        zrem = rpt - nfull_z * ch
        if zrem:
            pltpu.sync_copy(rows0.at[pl.ds(0, zrem)],
                            agg_sh.at[pl.ds(sid * rpt + nfull_z * ch, zrem)])

        @pl.when(sid == _NS - 1)
        def _():
            pltpu.sync_copy(rows0.at[pl.ds(0, nrem)],
                            agg_sh.at[pl.ds(_NS * rpt, nrem)])

        plsc.subcore_barrier()

        def idx_load(c, s):
            base = base0 + c * ch
            pltpu.async_copy(src_hbm.at[pl.ds(base, ch)], srcb[s], isem[s])
            pltpu.async_copy(dst_hbm.at[pl.ds(base, ch)], dstb[s], isem[s])

        def idx_wait(s):
            pltpu.make_async_copy(src_hbm.at[pl.ds(0, ch)], srcb[s],
                                  isem[s]).wait()
            pltpu.make_async_copy(dst_hbm.at[pl.ds(0, ch)], dstb[s],
                                  isem[s]).wait()

        def drain(b, s):
            pltpu.make_async_copy(rowsb[b], agg_sh.at[dstb[s]],
                                  ssem[b]).wait()

        def body(c, b2, s4, first):
            if not first:
                drain(b2, (s4 + 2) % 4)
            idx_load_c2 = c + 2

            @pl.when(idx_load_c2 < n_full)
            def _():
                idx_load(idx_load_c2, (s4 + 2) % 4)

            idx_wait(s4)
            pltpu.async_copy(h_hbm.at[srcb[s4]], rowsb[b2], gsem).wait()
            pltpu.async_copy(rowsb[b2], agg_sh.at[dstb[s4]], ssem[b2],
                             add=True)

        idx_load(0, 0)
        idx_load(1, 1)

        def quad(i, _):
            for b in range(4):
                c = 4 * i + b

                @pl.when(i == 0)
                def _():
                    if b < 2:
                        body(c, b % 2, b % 4, True)
                    else:
                        body(c, b % 2, b % 4, False)

                @pl.when(i > 0)
                def _():
                    body(c, b % 2, b % 4, False)
            return 0

        lax.fori_loop(0, n_full // 4, quad, 0)
        for c in range(n_full // 4 * 4, n_full):
            body(c, c % 2, c % 4, False)
        drain(0, (n_full - 2) % 4)
        drain(1, (n_full - 1) % 4)

        base = base0 + n_full * ch
        pltpu.sync_copy(src_hbm.at[pl.ds(base, tail)], src_t)
        pltpu.sync_copy(dst_hbm.at[pl.ds(base, tail)], dst_t)
        pltpu.async_copy(h_hbm.at[src_t], rows_t, gsem).wait()
        pltpu.sync_copy(rows_t, agg_sh.at[dst_t], add=True)

        plsc.subcore_barrier()

        pltpu.sync_copy(
            agg_sh.at[pl.ds(sid * rpt, rpt)],
            out_hbm.at[pl.ds(cid * n + sid * rpt, rpt)],
        )

        @pl.when(sid == _NS - 1)
        def _():
            pltpu.sync_copy(
                agg_sh.at[pl.ds(_NS * rpt, nrem)],
                out_hbm.at[pl.ds(cid * n + _NS * rpt, nrem)],
            )

    return k(h, src, dst)


def _tc_layer(h, agg, batch, p, *, n, d, hdim, g):
    eps = 1e-5

    def body(h_ref, agg_ref, b_ref, w1, b1, g1, be1, w2, b2, g2, be2,
             hout_ref, pool_ref):
        y = h_ref[...] + agg_ref[pl.ds(0, n), :] + agg_ref[pl.ds(n, n), :]
        z = jnp.dot(y, w1[...], preferred_element_type=jnp.float32) + b1[...]
        m = jnp.mean(z, axis=0)
        v = jnp.mean(z * z, axis=0) - m * m
        z = g1[...] * (z - m) * lax.rsqrt(v + eps) + be1[...]
        z = jnp.maximum(z, 0.0)
        z = jnp.dot(z, w2[...], preferred_element_type=jnp.float32) + b2[...]
        m2 = jnp.mean(z, axis=0)
        v2 = jnp.mean(z * z, axis=0) - m2 * m2
        z = g2[...] * (z - m2) * lax.rsqrt(v2 + eps) + be2[...]
        hn = jnp.maximum(z, 0.0)
        hout_ref[...] = hn
        seg = lax.broadcasted_iota(jnp.int32, (g, n), 0)
        onehot = (seg == b_ref[...][None, :]).astype(jnp.float32)
        pool_ref[...] = jnp.dot(onehot, hn, preferred_element_type=jnp.float32)

    return pl.pallas_call(
        body,
        out_shape=(
            jax.ShapeDtypeStruct((n, hdim), jnp.float32),
            jax.ShapeDtypeStruct((g, hdim), jnp.float32),
        ),
    )(h, agg, batch, p["W1"], p["b1"], p["g1"], p["be1"],
      p["W2"], p["b2"], p["g2"], p["be2"])


def kernel(x, edge_index, batch, params):
    n, d = x.shape
    e = edge_index.shape[1]
    g = 64
    src = edge_index[0]
    dst = edge_index[1]
    h = x
    pooled = []
    for p in params:
        hdim = p["W2"].shape[1]
        agg = _sc_scatter_add(h, src, dst, n=n, e=e, d=h.shape[1])
        h, pool = _tc_layer(h, agg, batch, p, n=n, d=h.shape[1], hdim=hdim, g=g)
        pooled.append(pool)
    return jnp.concatenate(pooled, axis=-1)


# speedup vs baseline: 12.8400x; 1.2353x over previous
import functools

import jax
import jax.numpy as jnp
from jax import lax
from jax.experimental import pallas as pl
from jax.experimental.pallas import tpu as pltpu
from jax.experimental.pallas import tpu_sc as plsc

_NC = 2
_NS = 16


@functools.partial(jax.jit, static_argnames=("n", "e", "d"))
def _sc_scatter_add(h, src, dst, *, n, e, d):
    nw = _NC * _NS
    epw = e // nw
    ch = 80
    nch = epw // ch
    assert nch * ch == epw
    NR = 4
    NI = 8
    rpt = (n // _NS) // 8 * 8
    nrem = n - _NS * rpt

    mesh = plsc.VectorSubcoreMesh(core_axis_name="c", subcore_axis_name="s")

    scratch = (
        [pltpu.VMEM((ch,), jnp.int32)] * NI +
        [pltpu.VMEM((ch,), jnp.int32)] * NI +
        [pltpu.VMEM((ch, d), jnp.float32)] * NR +
        [pltpu.VMEM_SHARED((n, d), jnp.float32)] +
        [pltpu.SemaphoreType.DMA] * NR +
        [pltpu.SemaphoreType.DMA] * NI +
        [pltpu.SemaphoreType.DMA] * NR
    )

    @functools.partial(
        pl.kernel,
        out_type=jax.ShapeDtypeStruct((_NC * n, d), jnp.float32),
        mesh=mesh,
        scratch_types=scratch,
    )
    def k(h_hbm, src_hbm, dst_hbm, out_hbm, *refs):
        srcb = refs[0:NI]
        dstb = refs[NI:2 * NI]
        rowsb = refs[2 * NI:2 * NI + NR]
        agg_sh = refs[2 * NI + NR]
        gsem = refs[2 * NI + NR + 1:2 * NI + 2 * NR + 1]
        isem = refs[2 * NI + 2 * NR + 1:3 * NI + 2 * NR + 1]
        ssem = refs[3 * NI + 2 * NR + 1:3 * NI + 3 * NR + 1]
        cid = lax.axis_index("c")
        sid = lax.axis_index("s")
        wid = cid * _NS + sid
        base0 = wid * epw

        zeros16 = jnp.zeros((16,), jnp.float32)
        rows0 = rowsb[0]

        def zrow(r, _):
            for j in range(d // 16):
                rows0[r, pl.ds(j * 16, 16)] = zeros16
            return 0

        lax.fori_loop(0, ch, zrow, 0)
        nfull_z = rpt // ch
        for i in range(nfull_z):
            pltpu.sync_copy(rows0, agg_sh.at[pl.ds(sid * rpt + i * ch, ch)])
        zrem = rpt - nfull_z * ch
        if zrem:
            pltpu.sync_copy(rows0.at[pl.ds(0, zrem)],
                            agg_sh.at[pl.ds(sid * rpt + nfull_z * ch, zrem)])

        @pl.when(sid == _NS - 1)
        def _():
            pltpu.sync_copy(rows0.at[pl.ds(0, nrem)],
                            agg_sh.at[pl.ds(_NS * rpt, nrem)])

        plsc.subcore_barrier()

        def idx_load(c, s):
            base = base0 + c * ch
            pltpu.async_copy(src_hbm.at[pl.ds(base, ch)], srcb[s], isem[s])
            pltpu.async_copy(dst_hbm.at[pl.ds(base, ch)], dstb[s], isem[s])

        def idx_wait(s):
            pltpu.make_async_copy(src_hbm.at[pl.ds(0, ch)], srcb[s],
                                  isem[s]).wait()
            pltpu.make_async_copy(dst_hbm.at[pl.ds(0, ch)], dstb[s],
                                  isem[s]).wait()

        def gather(c_r, c_s):
            pltpu.async_copy(h_hbm.at[srcb[c_s]], rowsb[c_r], gsem[c_r])

        def gather_wait(c_r, c_s):
            pltpu.make_async_copy(h_hbm.at[srcb[c_s]], rowsb[c_r],
                                  gsem[c_r]).wait()

        def drain(c_r, c_s):
            pltpu.make_async_copy(rowsb[c_r], agg_sh.at[dstb[c_s]],
                                  ssem[c_r]).wait()

        def body(c, cm, *, do_drain, do_idx, do_gather):
            r, s = cm % NR, cm % NI
            if do_drain:
                drain((cm + 2) % NR, (cm + 6) % NI)
            if do_idx is None:
                @pl.when(c + 4 < nch)
                def _():
                    idx_load(c + 4, (cm + 4) % NI)
            elif do_idx:
                idx_load(c + 4, (cm + 4) % NI)
            if do_gather:
                idx_wait((cm + 2) % NI)
                gather((cm + 2) % NR, (cm + 2) % NI)
            gather_wait(r, s)
            pltpu.async_copy(rowsb[r], agg_sh.at[dstb[s]], ssem[r], add=True)

        for c0 in range(4):
            idx_load(c0, c0)
        for c0 in range(2):
            idx_wait(c0)
            gather(c0, c0)
        body(0, 0, do_drain=False, do_idx=True, do_gather=True)
        body(1, 1, do_drain=False, do_idx=True, do_gather=True)

        def octet(i, _):
            for b in range(8):
                c = 8 * i + 2 + b
                body(c, 2 + b, do_drain=True, do_idx=None, do_gather=True)
            return 0

        lax.fori_loop(0, (nch - 5) // 8, octet, 0)
        body(nch - 3, nch - 3, do_drain=True, do_idx=False, do_gather=True)
        body(nch - 2, nch - 2, do_drain=True, do_idx=False, do_gather=False)
        body(nch - 1, nch - 1, do_drain=True, do_idx=False, do_gather=False)
        drain((nch - 2) % NR, (nch - 2) % NI)
        drain((nch - 1) % NR, (nch - 1) % NI)

        plsc.subcore_barrier()

        pltpu.sync_copy(
            agg_sh.at[pl.ds(sid * rpt, rpt)],
            out_hbm.at[pl.ds(cid * n + sid * rpt, rpt)],
        )

        @pl.when(sid == _NS - 1)
        def _():
            pltpu.sync_copy(
                agg_sh.at[pl.ds(_NS * rpt, nrem)],
                out_hbm.at[pl.ds(cid * n + _NS * rpt, nrem)],
            )

    return k(h, src, dst)


def _tc_layer(h, agg, batch, p, *, n, d, hdim, g):
    eps = 1e-5

    def body(h_ref, agg_ref, b_ref, w1, b1, g1, be1, w2, b2, g2, be2,
             hout_ref, pool_ref):
        y = h_ref[...] + agg_ref[pl.ds(0, n), :] + agg_ref[pl.ds(n, n), :]
        z = jnp.dot(y, w1[...], preferred_element_type=jnp.float32) + b1[...]
        m = jnp.mean(z, axis=0)
        v = jnp.mean(z * z, axis=0) - m * m
        z = g1[...] * (z - m) * lax.rsqrt(v + eps) + be1[...]
        z = jnp.maximum(z, 0.0)
        z = jnp.dot(z, w2[...], preferred_element_type=jnp.float32) + b2[...]
        m2 = jnp.mean(z, axis=0)
        v2 = jnp.mean(z * z, axis=0) - m2 * m2
        z = g2[...] * (z - m2) * lax.rsqrt(v2 + eps) + be2[...]
        hn = jnp.maximum(z, 0.0)
        hout_ref[...] = hn
        seg = lax.broadcasted_iota(jnp.int32, (g, n), 0)
        onehot = (seg == b_ref[...][None, :]).astype(jnp.float32)
        pool_ref[...] = jnp.dot(onehot, hn, preferred_element_type=jnp.float32)

    return pl.pallas_call(
        body,
        out_shape=(
            jax.ShapeDtypeStruct((n, hdim), jnp.float32),
            jax.ShapeDtypeStruct((g, hdim), jnp.float32),
        ),
    )(h, agg, batch, p["W1"], p["b1"], p["g1"], p["be1"],
      p["W2"], p["b2"], p["g2"], p["be2"])


def kernel(x, edge_index, batch, params):
    n, d = x.shape
    e = edge_index.shape[1]
    g = 64
    src = edge_index[0]
    dst = edge_index[1]
    h = x
    pooled = []
    for p in params:
        hdim = p["W2"].shape[1]
        agg = _sc_scatter_add(h, src, dst, n=n, e=e, d=h.shape[1])
        h, pool = _tc_layer(h, agg, batch, p, n=n, d=h.shape[1], hdim=hdim, g=g)
        pooled.append(pool)
    return jnp.concatenate(pooled, axis=-1)
